# baseline (speedup 1.0000x reference)
import dataclasses

import jax
import jax.numpy as jnp
from jax import lax
from jax.experimental import pallas as pl
from jax.experimental.pallas import tpu as pltpu
from jax.experimental.pallas import tpu_sc as plsc

EMB = 512
ATT = 1024
TOPB = 32
N = 16384
G = 64
BLK = 1024
NBLK = N // BLK

_BF = jnp.bfloat16
_CH = 2048
_NPAD = N + _CH
_LANES = 16
_NEG = jnp.float32(-jnp.inf)


def _mdot(a, b):
    return jax.lax.dot_general(a, b, (((1,), (0,)), ((), ())),
                               preferred_element_type=jnp.float32)


def _score_body(x_ref, batch_ref, graph_ref, proto_ref, W1_ref, b1_ref,
                W2_ref, b2_ref, Ws_ref, bs_ref, out_ref, counts_ref):
    xb = x_ref[...].astype(_BF)
    b = batch_ref[0, 0, :]
    giota = jax.lax.broadcasted_iota(jnp.int32, (BLK, G), 1)
    onehot = b[:, None] == giota
    molb = _mdot(onehot.astype(_BF), graph_ref[...].astype(_BF)).astype(_BF)
    pc = jnp.mean(proto_ref[...], axis=0, keepdims=True).astype(_BF)
    pb = jnp.broadcast_to(pc, (BLK, EMB))
    sf = jnp.concatenate([xb, molb, pb], axis=-1)
    pre = _mdot(sf, W1_ref[...]) + b1_ref[...]
    h = jnp.maximum(pre, 0.0)
    hb = h.astype(_BF)
    h2 = _mdot(hb, W2_ref[...]) + b2_ref[...]
    out_ref[...] = _mdot(h2, Ws_ref[...]) + bs_ref[...]

    bc = jnp.sum(onehot.astype(jnp.int32), axis=0)[None, :]

    @pl.when(pl.program_id(0) == 0)
    def _():
        counts_ref[...] = bc

    @pl.when(pl.program_id(0) != 0)
    def _():
        counts_ref[...] = counts_ref[...] + bc


def _sc_topk_body(counts_hbm, scores_hbm, node_hbm, out_hbm,
                  cnt_v, sbuf, idxbuf, wbuf, rows_v, acc, sem):
    core = jax.lax.axis_index("c")
    sub = jax.lax.axis_index("s")
    wid = sub * 2 + core
    pltpu.sync_copy(counts_hbm, cnt_v)
    iota = jax.lax.broadcasted_iota(jnp.int32, (_LANES,), 0)
    cchunks = [cnt_v[pl.ds(k * _LANES, _LANES)] for k in range(G // _LANES)]

    for j in range(2):
        g = wid * 2 + j
        start = jnp.int32(0)
        n = jnp.int32(0)
        for k in range(G // _LANES):
            pos = k * _LANES + iota
            start = start + jnp.sum(jnp.where(pos < g, cchunks[k], 0))
            n = n + jnp.sum(jnp.where(pos == g, cchunks[k], 0))

        @pl.loop(0, EMB // _LANES)
        def _(d):
            acc[pl.ds(d * _LANES, _LANES)] = jnp.zeros((_LANES,), jnp.float32)

        @pl.when(n > 0)
        def _():
            astart = (start // 8) * 8
            lo = start - astart
            hi = lo + n
            nch = (hi + _CH - 1) // _CH

            @pl.loop(0, nch)
            def _(i):
                pltpu.sync_copy(
                    scores_hbm.at[pl.ds(astart + i * _CH, _CH)],
                    sbuf.at[pl.ds(i * _CH, _CH)])

            nc16 = (hi + _LANES - 1) // _LANES

            def max_fn(c, mv):
                off = c * _LANES
                v = sbuf[pl.ds(off, _LANES)]
                pos = off + iota
                v = jnp.where((pos >= lo) & (pos < hi), v, _NEG)
                sbuf[pl.ds(off, _LANES)] = v
                return jnp.maximum(mv, v)

            mvec = lax.fori_loop(0, nc16, max_fn,
                                 jnp.full((_LANES,), _NEG, jnp.float32))
            m = jnp.max(mvec)

            def den_fn(c, dv):
                v = sbuf[pl.ds(c * _LANES, _LANES)]
                e = jnp.where(v > _NEG, jnp.exp(v - m), 0.0)
                return dv + e

            dvec = lax.fori_loop(0, nc16, den_fn,
                                 jnp.zeros((_LANES,), jnp.float32))
            denom = jnp.sum(dvec)
            denom = jnp.where(denom > 0.0, denom, 1.0)

            def round_fn(r, carry):
                w_lo, w_hi, i_lo, i_hi = carry

                def scan_fn(c, bc_):
                    best, besti = bc_
                    off = c * _LANES
                    v = sbuf[pl.ds(off, _LANES)]
                    upd = v > best
                    return (jnp.maximum(best, v),
                            jnp.where(upd, off + iota, besti))

                best, besti = lax.fori_loop(
                    0, nc16, scan_fn,
                    (jnp.full((_LANES,), _NEG, jnp.float32),
                     jnp.zeros((_LANES,), jnp.int32)))
                gmax = jnp.max(best)
                cand = jnp.where(best == gmax, besti, jnp.int32(_NPAD))
                gidx = jnp.min(cand)
                gidx = jnp.where(gidx >= _NPAD, 0, gidx)
                wv = jnp.exp(jnp.full((_LANES,), gmax, jnp.float32) - m) / denom
                sel_lo = iota == r
                sel_hi = iota == r - _LANES
                iv = jnp.full((_LANES,), astart + gidx, jnp.int32)
                w_lo = jnp.where(sel_lo, wv, w_lo)
                w_hi = jnp.where(sel_hi, wv, w_hi)
                i_lo = jnp.where(sel_lo, iv, i_lo)
                i_hi = jnp.where(sel_hi, iv, i_hi)
                cs = (gidx // _LANES) * _LANES
                vch = sbuf[pl.ds(cs, _LANES)]
                sbuf[pl.ds(cs, _LANES)] = jnp.where(iota == gidx - cs,
                                                    _NEG, vch)
                return w_lo, w_hi, i_lo, i_hi

            zf = jnp.zeros((_LANES,), jnp.float32)
            zi = jnp.zeros((_LANES,), jnp.int32)
            w_lo, w_hi, i_lo, i_hi = lax.fori_loop(
                0, TOPB, round_fn, (zf, zf, zi, zi))
            wbuf[pl.ds(0, _LANES)] = w_lo
            wbuf[pl.ds(_LANES, _LANES)] = w_hi
            idxbuf[pl.ds(0, _LANES)] = i_lo
            idxbuf[pl.ds(_LANES, _LANES)] = i_hi

            pltpu.async_copy(node_hbm.at[idxbuf], rows_v, sem).wait()

            @pl.loop(0, TOPB)
            def _(r):
                base = (r // _LANES) * _LANES
                wchunk = wbuf[pl.ds(base, _LANES)]
                w = jnp.sum(jnp.where(iota == r - base, wchunk, 0.0))

                @pl.loop(0, EMB // _LANES)
                def _(dd):
                    sl = pl.ds(dd * _LANES, _LANES)
                    acc[sl] = acc[sl] + rows_v[r, sl] * w

        pltpu.sync_copy(acc, out_hbm.at[g])


def kernel(node_repr, graph_repr, prototypes, batch, W1, b1, W2, b2, Ws, bs):
    batch = batch.astype(jnp.int32)
    batch3 = batch.reshape(NBLK, 1, BLK)
    scores, counts = pl.pallas_call(
        _score_body,
        grid=(NBLK,),
        in_specs=[
            pl.BlockSpec((BLK, EMB), lambda i: (i, 0)),
            pl.BlockSpec((1, 1, BLK), lambda i: (i, 0, 0)),
            pl.BlockSpec((G, EMB), lambda i: (0, 0)),
            pl.BlockSpec((2, EMB), lambda i: (0, 0)),
            pl.BlockSpec((3 * EMB, ATT), lambda i: (0, 0)),
            pl.BlockSpec((1, ATT), lambda i: (0, 0)),
            pl.BlockSpec((ATT, ATT), lambda i: (0, 0)),
            pl.BlockSpec((1, ATT), lambda i: (0, 0)),
            pl.BlockSpec((ATT, 1), lambda i: (0, 0)),
            pl.BlockSpec((1, 1), lambda i: (0, 0)),
        ],
        out_specs=[
            pl.BlockSpec((BLK, 1), lambda i: (i, 0)),
            pl.BlockSpec((1, G), lambda i: (0, 0)),
        ],
        out_shape=[
            jax.ShapeDtypeStruct((N, 1), jnp.float32),
            jax.ShapeDtypeStruct((1, G), jnp.int32),
        ],
    )(node_repr, batch3, graph_repr, prototypes, W1, b1.reshape(1, ATT),
      W2, b2.reshape(1, ATT), Ws, bs.reshape(1, 1))

    scores_pad = jnp.pad(scores.reshape(N), (0, _NPAD - N))
    counts1 = counts.reshape(G)

    cp = pltpu.CompilerParams()
    if "needs_layout_passes" in pltpu.CompilerParams.__dataclass_fields__:
        cp = dataclasses.replace(cp, needs_layout_passes=False)
    mesh = plsc.VectorSubcoreMesh(core_axis_name="c", subcore_axis_name="s")
    sc_topk = pl.kernel(
        _sc_topk_body,
        mesh=mesh,
        compiler_params=cp,
        out_type=jax.ShapeDtypeStruct((G, EMB), jnp.float32),
        scratch_types=[
            pltpu.VMEM((G,), jnp.int32),
            pltpu.VMEM((N,), jnp.float32),
            pltpu.VMEM((TOPB,), jnp.int32),
            pltpu.VMEM((TOPB,), jnp.float32),
            pltpu.VMEM((TOPB, EMB), jnp.float32),
            pltpu.VMEM((EMB,), jnp.float32),
            pltpu.SemaphoreType.DMA,
        ],
    )
    return sc_topk(counts1, scores_pad, node_repr)
